# baseline (speedup 1.0000x reference)
import functools

import jax
import jax.numpy as jnp
from jax import lax
from jax.experimental import pallas as pl
from jax.experimental.pallas import tpu as pltpu
from jax.experimental.pallas import tpu_sc as plsc

D_M = 512
D_H = 64
ALPHA = 2048
L_OBS = 1024

_NC = 2
_NS = 16
_NW = _NC * _NS
_FPW = D_M // _NW
_LANES = 16
_CHUNKS = ALPHA // _LANES


_OCHUNKS = L_OBS // _LANES


def _impute_body(t_hbm, x_hbm, g_hbm, out_hbm,
                 times_v, obs_v, rows_v, bins_v, pos_v, pref_v, pref2_v,
                 dma_sem, out_sem):
    wid = lax.axis_index("s") * _NC + lax.axis_index("c")
    f0 = wid * _FPW
    copies = []
    for j in range(_FPW):
        copies.append(pltpu.async_copy(
            t_hbm.at[f0 + j], times_v.at[pl.ds(j * L_OBS, L_OBS)], dma_sem))
        copies.append(pltpu.async_copy(
            x_hbm.at[f0 + j], obs_v.at[pl.ds(j * L_OBS, L_OBS)], dma_sem))
    copies.append(pltpu.async_copy(
        g_hbm.at[pl.ds(f0, _FPW)],
        obs_v.at[pl.ds(_FPW * L_OBS, _FPW)], dma_sem))
    for cp in copies:
        cp.wait()

    lanes = lax.iota(jnp.int32, _LANES)
    zero_v = jnp.zeros((_LANES,), jnp.int32)

    @plsc.parallel_loop(0, _CHUNKS, unroll=4)
    def zero_bins(c):
        bins_v[pl.ds(c * _LANES, _LANES)] = zero_v

    def per_feature(j, carry):
        bt = j * L_OBS
        bo = j * ALPHA

        @plsc.parallel_loop(0, _OCHUNKS, unroll=4)
        def scatter_pass(c):
            li1 = lanes + (bt + c * _LANES + 1)
            tau = times_v[pl.ds(bt + c * _LANES, _LANES)]
            tau_nxt = plsc.load_gather(times_v, [li1])
            mask = (tau != tau_nxt) | (li1 == bt + L_OBS)
            plsc.store_scatter(bins_v, [tau], li1, mask=mask)

        @plsc.parallel_loop(0, _CHUNKS, unroll=4)
        def local_scan(c):
            v = bins_v[pl.ds(c * _LANES, _LANES)]
            bins_v[pl.ds(c * _LANES, _LANES)] = zero_v
            pos_v[pl.ds(c * _LANES, _LANES)] = plsc.cummax(v)

        tails = lanes * _LANES + (_LANES - 1)

        def group_scan(g, carry_vec):
            tot = plsc.load_gather(pos_v, [tails + g * (_LANES * _LANES)])
            pm = jnp.maximum(plsc.cummax(tot), carry_vec)
            pref_v[pl.ds(g * _LANES, _LANES)] = pm
            return plsc.load_gather(
                pref_v, [jnp.full((_LANES,), g * _LANES + (_LANES - 1),
                                  jnp.int32)])

        lax.fori_loop(0, _CHUNKS // _LANES, group_scan, zero_v)

        @plsc.parallel_loop(0, _CHUNKS // _LANES)
        def excl_pass(g):
            gi = lanes + g * _LANES
            e = plsc.load_gather(pref_v, [jnp.maximum(gi - 1, 0)])
            pref2_v[pl.ds(g * _LANES, _LANES)] = jnp.where(gi > 0, e, 0)

        @plsc.parallel_loop(0, _CHUNKS, unroll=4)
        def fill_pass(c):
            s = pos_v[pl.ds(c * _LANES, _LANES)]
            p = plsc.load_gather(pref2_v, [lax.broadcast(c, (_LANES,))])
            pos = jnp.maximum(s, p)
            gidx = jnp.where(pos > 0, pos - 1, _FPW * L_OBS + j)
            rows_v[pl.ds(bo + c * _LANES, _LANES)] = plsc.load_gather(
                obs_v, [gidx])

        pltpu.async_copy(
            rows_v.at[pl.ds(bo, ALPHA)], out_hbm.at[f0 + j], out_sem)
        return carry

    lax.fori_loop(0, _FPW, per_feature, jnp.int32(0))
    for j in range(_FPW):
        pltpu.make_async_copy(
            rows_v.at[pl.ds(j * ALPHA, ALPHA)], out_hbm.at[f0 + j],
            out_sem).wait()


_impute_sc = functools.partial(
    pl.kernel,
    out_type=jax.ShapeDtypeStruct((D_M, ALPHA), jnp.float32),
    mesh=plsc.VectorSubcoreMesh(
        core_axis_name="c", subcore_axis_name="s",
        num_cores=_NC, num_subcores=_NS),
    compiler_params=pltpu.CompilerParams(needs_layout_passes=False),
    scratch_types=[
        pltpu.VMEM((_FPW * L_OBS + _LANES,), jnp.int32),
        pltpu.VMEM((_FPW * L_OBS + _FPW,), jnp.float32),
        pltpu.VMEM((_FPW * ALPHA,), jnp.float32),
        pltpu.VMEM((ALPHA,), jnp.int32),
        pltpu.VMEM((ALPHA,), jnp.int32),
        pltpu.VMEM((_CHUNKS,), jnp.int32),
        pltpu.VMEM((_CHUNKS,), jnp.int32),
        pltpu.SemaphoreType.DMA,
        pltpu.SemaphoreType.DMA,
    ],
)(_impute_body)


def _matmul_body(rs_ref, w_ref, b_ref, out_ref):
    out_ref[...] = lax.dot_general(
        rs_ref[...], w_ref[...], (((0,), (1,)), ((), ())),
        preferred_element_type=jnp.float32) + b_ref[...]


_A_BLK = 2048


def _matmul_tc(rs, W, b2):
    return pl.pallas_call(
        _matmul_body,
        grid=(ALPHA // _A_BLK,),
        in_specs=[
            pl.BlockSpec((D_M, _A_BLK), lambda k: (0, k)),
            pl.BlockSpec((D_H, D_M), lambda k: (0, 0)),
            pl.BlockSpec((1, D_H), lambda k: (0, 0)),
        ],
        out_specs=pl.BlockSpec((_A_BLK, D_H), lambda k: (k, 0)),
        out_shape=jax.ShapeDtypeStruct((ALPHA, D_H), jnp.float32),
    )(rs, W, b2)


def kernel(x_ts, t_ts, global_means, W, b):
    rs = _impute_sc(t_ts.astype(jnp.int32), x_ts, global_means)
    return _matmul_tc(rs, W, b.reshape(1, D_H))


# speedup vs baseline: 1.0357x; 1.0357x over previous
import functools

import jax
import jax.numpy as jnp
from jax import lax
from jax.experimental import pallas as pl
from jax.experimental.pallas import tpu as pltpu
from jax.experimental.pallas import tpu_sc as plsc

D_M = 512
D_H = 64
ALPHA = 2048
L_OBS = 1024

_NC = 2
_NS = 16
_NW = _NC * _NS
_FPW = D_M // _NW
_LANES = 16
_CHUNKS = ALPHA // _LANES


_OCHUNKS = L_OBS // _LANES


def _impute_body(t_hbm, x_hbm, g_hbm, out_hbm,
                 times_v, obs_v, rows_v, bins_v, pos_v, pref_v, pref2_v,
                 dma_sem, out_sem):
    wid = lax.axis_index("s") * _NC + lax.axis_index("c")
    f0 = wid * _FPW
    copies = []
    for j in range(_FPW):
        copies.append(pltpu.async_copy(
            t_hbm.at[f0 + j], times_v.at[pl.ds(j * L_OBS, L_OBS)], dma_sem))
        copies.append(pltpu.async_copy(
            x_hbm.at[f0 + j], obs_v.at[pl.ds(j * L_OBS, L_OBS)], dma_sem))
    copies.append(pltpu.async_copy(
        g_hbm.at[pl.ds(f0, _FPW)],
        obs_v.at[pl.ds(_FPW * L_OBS, _FPW)], dma_sem))
    for cp in copies:
        cp.wait()

    lanes = lax.iota(jnp.int32, _LANES)
    zero_v = jnp.zeros((_LANES,), jnp.int32)

    @plsc.parallel_loop(0, _CHUNKS, unroll=4)
    def zero_bins(c):
        bins_v[pl.ds(c * _LANES, _LANES)] = zero_v

    def per_feature(j, carry):
        bt = j * L_OBS
        bo = j * ALPHA

        @plsc.parallel_loop(0, _OCHUNKS, unroll=4)
        def scatter_pass(c):
            li1 = lanes + (bt + c * _LANES + 1)
            tau = times_v[pl.ds(bt + c * _LANES, _LANES)]
            tau_nxt = plsc.load_gather(times_v, [li1])
            mask = (tau != tau_nxt) | (li1 == bt + L_OBS)
            plsc.store_scatter(bins_v, [tau], li1, mask=mask)

        @plsc.parallel_loop(0, _CHUNKS, unroll=4)
        def local_scan(c):
            v = bins_v[pl.ds(c * _LANES, _LANES)]
            bins_v[pl.ds(c * _LANES, _LANES)] = zero_v
            pos_v[pl.ds(c * _LANES, _LANES)] = plsc.cummax(v)

        tails = lanes * _LANES + (_LANES - 1)

        def group_scan(g, carry_vec):
            tot = plsc.load_gather(pos_v, [tails + g * (_LANES * _LANES)])
            pm = jnp.maximum(plsc.cummax(tot), carry_vec)
            pref_v[pl.ds(g * _LANES, _LANES)] = pm
            return plsc.load_gather(
                pref_v, [jnp.full((_LANES,), g * _LANES + (_LANES - 1),
                                  jnp.int32)])

        lax.fori_loop(0, _CHUNKS // _LANES, group_scan, zero_v)

        @plsc.parallel_loop(0, _CHUNKS // _LANES)
        def excl_pass(g):
            gi = lanes + g * _LANES
            e = plsc.load_gather(pref_v, [jnp.maximum(gi - 1, 0)])
            pref2_v[pl.ds(g * _LANES, _LANES)] = jnp.where(gi > 0, e, 0)

        @plsc.parallel_loop(0, _CHUNKS // 2, unroll=4)
        def fill_pass(c):
            t_even = lanes * 2 + c * (2 * _LANES)
            t_odd = t_even + 1
            s_e = plsc.load_gather(pos_v, [t_even])
            s_o = plsc.load_gather(pos_v, [t_odd])
            p = plsc.load_gather(
                pref2_v, [lax.shift_right_logical(t_even, 4)])
            pos_e = jnp.maximum(s_e, p)
            pos_o = jnp.maximum(s_o, p)
            gid_e = jnp.where(pos_e > 0, pos_e - 1, _FPW * L_OBS + j)
            gid_o = jnp.where(pos_o > 0, pos_o - 1, _FPW * L_OBS + j)
            val_e = plsc.load_gather(obs_v, [gid_e])
            val_o = plsc.load_gather(obs_v, [gid_o])
            packed = plsc.pack(val_e, val_o,
                               format=plsc.PackFormat.INTERLEAVED)
            rows_v[j, pl.ds(c * (2 * _LANES), 2 * _LANES)] = packed

        @pl.when(j == _FPW // 2 - 1)
        def _():
            pltpu.async_copy(
                rows_v.at[pl.ds(0, _FPW // 2)],
                out_hbm.at[pl.ds(f0, _FPW // 2)], out_sem)

        return carry

    lax.fori_loop(0, _FPW, per_feature, jnp.int32(0))
    pltpu.async_copy(
        rows_v.at[pl.ds(_FPW // 2, _FPW // 2)],
        out_hbm.at[pl.ds(f0 + _FPW // 2, _FPW // 2)], out_sem)
    pltpu.make_async_copy(
        rows_v.at[pl.ds(0, _FPW // 2)],
        out_hbm.at[pl.ds(f0, _FPW // 2)], out_sem).wait()
    pltpu.make_async_copy(
        rows_v.at[pl.ds(_FPW // 2, _FPW // 2)],
        out_hbm.at[pl.ds(f0 + _FPW // 2, _FPW // 2)], out_sem).wait()


_impute_sc = functools.partial(
    pl.kernel,
    out_type=jax.ShapeDtypeStruct((D_M, ALPHA), jnp.bfloat16),
    mesh=plsc.VectorSubcoreMesh(
        core_axis_name="c", subcore_axis_name="s",
        num_cores=_NC, num_subcores=_NS),
    compiler_params=pltpu.CompilerParams(needs_layout_passes=False),
    scratch_types=[
        pltpu.VMEM((_FPW * L_OBS + _LANES,), jnp.int32),
        pltpu.VMEM((_FPW * L_OBS + _FPW,), jnp.float32),
        pltpu.VMEM((_FPW, ALPHA), jnp.bfloat16),
        pltpu.VMEM((ALPHA,), jnp.int32),
        pltpu.VMEM((ALPHA,), jnp.int32),
        pltpu.VMEM((_CHUNKS,), jnp.int32),
        pltpu.VMEM((_CHUNKS,), jnp.int32),
        pltpu.SemaphoreType.DMA,
        pltpu.SemaphoreType.DMA,
    ],
)(_impute_body)


def _matmul_body(rs_ref, w_ref, b_ref, out_ref):
    out_ref[...] = lax.dot_general(
        rs_ref[...], w_ref[...], (((0,), (1,)), ((), ())),
        preferred_element_type=jnp.float32) + b_ref[...]


_A_BLK = 1024


def _matmul_tc(rs, W, b2):
    return pl.pallas_call(
        _matmul_body,
        grid=(ALPHA // _A_BLK,),
        in_specs=[
            pl.BlockSpec((D_M, _A_BLK), lambda k: (0, k)),
            pl.BlockSpec((D_H, D_M), lambda k: (0, 0)),
            pl.BlockSpec((1, D_H), lambda k: (0, 0)),
        ],
        out_specs=pl.BlockSpec((_A_BLK, D_H), lambda k: (k, 0)),
        out_shape=jax.ShapeDtypeStruct((ALPHA, D_H), jnp.float32),
    )(rs, W, b2)


def kernel(x_ts, t_ts, global_means, W, b):
    rs = _impute_sc(t_ts.astype(jnp.int32), x_ts, global_means)
    return _matmul_tc(rs, W.astype(jnp.bfloat16), b.reshape(1, D_H))


# speedup vs baseline: 1.0442x; 1.0083x over previous
import functools

import jax
import jax.numpy as jnp
from jax import lax
from jax.experimental import pallas as pl
from jax.experimental.pallas import tpu as pltpu
from jax.experimental.pallas import tpu_sc as plsc

D_M = 512
D_H = 64
ALPHA = 2048
L_OBS = 1024

_NC = 2
_NS = 16
_NW = _NC * _NS
_FPW = D_M // _NW
_LANES = 16
_CHUNKS = ALPHA // _LANES


_OCHUNKS = L_OBS // _LANES


def _impute_body(t_hbm, x_hbm, g_hbm, out_hbm,
                 times_v, obs_v, rows_v, bins_v, pos_v, pref_v, pref2_v,
                 dma_sem, rest_sem, out_sem):
    wid = lax.axis_index("s") * _NC + lax.axis_index("c")
    f0 = wid * _FPW
    head = [
        pltpu.async_copy(
            t_hbm.at[f0], times_v.at[pl.ds(0, L_OBS)], dma_sem),
        pltpu.async_copy(
            x_hbm.at[f0], obs_v.at[pl.ds(0, L_OBS)], dma_sem),
        pltpu.async_copy(
            g_hbm.at[pl.ds(f0, _FPW)],
            obs_v.at[pl.ds(_FPW * L_OBS, _FPW)], dma_sem),
    ]
    rest = []
    for j in range(1, _FPW):
        rest.append(pltpu.async_copy(
            t_hbm.at[f0 + j], times_v.at[pl.ds(j * L_OBS, L_OBS)], rest_sem))
        rest.append(pltpu.async_copy(
            x_hbm.at[f0 + j], obs_v.at[pl.ds(j * L_OBS, L_OBS)], rest_sem))
    for cp in head:
        cp.wait()

    lanes = lax.iota(jnp.int32, _LANES)
    zero_v = jnp.zeros((_LANES,), jnp.int32)

    @plsc.parallel_loop(0, _CHUNKS, unroll=4)
    def zero_bins(c):
        bins_v[pl.ds(c * _LANES, _LANES)] = zero_v

    def per_feature(j, carry):
        bt = j * L_OBS
        bo = j * ALPHA

        @plsc.parallel_loop(0, _OCHUNKS, unroll=4)
        def scatter_pass(c):
            li1 = lanes + (bt + c * _LANES + 1)
            tau = times_v[pl.ds(bt + c * _LANES, _LANES)]
            tau_nxt = plsc.load_gather(times_v, [li1])
            mask = (tau != tau_nxt) | (li1 == bt + L_OBS)
            plsc.store_scatter(bins_v, [tau], li1, mask=mask)

        @plsc.parallel_loop(0, _CHUNKS, unroll=4)
        def local_scan(c):
            v = bins_v[pl.ds(c * _LANES, _LANES)]
            bins_v[pl.ds(c * _LANES, _LANES)] = zero_v
            pos_v[pl.ds(c * _LANES, _LANES)] = plsc.cummax(v)

        tails = lanes * _LANES + (_LANES - 1)

        def group_scan(g, carry_vec):
            tot = plsc.load_gather(pos_v, [tails + g * (_LANES * _LANES)])
            pm = jnp.maximum(plsc.cummax(tot), carry_vec)
            pref_v[pl.ds(g * _LANES, _LANES)] = pm
            return plsc.load_gather(
                pref_v, [jnp.full((_LANES,), g * _LANES + (_LANES - 1),
                                  jnp.int32)])

        lax.fori_loop(0, _CHUNKS // _LANES, group_scan, zero_v)

        @plsc.parallel_loop(0, _CHUNKS // _LANES)
        def excl_pass(g):
            gi = lanes + g * _LANES
            e = plsc.load_gather(pref_v, [jnp.maximum(gi - 1, 0)])
            pref2_v[pl.ds(g * _LANES, _LANES)] = jnp.where(gi > 0, e, 0)

        @plsc.parallel_loop(0, _CHUNKS // 2, unroll=4)
        def fill_pass(c):
            t_even = lanes * 2 + c * (2 * _LANES)
            t_odd = t_even + 1
            s_e = plsc.load_gather(pos_v, [t_even])
            s_o = plsc.load_gather(pos_v, [t_odd])
            p = plsc.load_gather(
                pref2_v, [lax.shift_right_logical(t_even, 4)])
            pos_e = jnp.maximum(s_e, p)
            pos_o = jnp.maximum(s_o, p)
            gid_e = jnp.where(pos_e > 0, pos_e - 1, _FPW * L_OBS + j)
            gid_o = jnp.where(pos_o > 0, pos_o - 1, _FPW * L_OBS + j)
            val_e = plsc.load_gather(obs_v, [gid_e])
            val_o = plsc.load_gather(obs_v, [gid_o])
            packed = plsc.pack(val_e, val_o,
                               format=plsc.PackFormat.INTERLEAVED)
            rows_v[j, pl.ds(c * (2 * _LANES), 2 * _LANES)] = packed

        @pl.when(jnp.equal(j, _FPW // 2 - 1))
        def _():
            pltpu.async_copy(
                rows_v.at[pl.ds(0, _FPW // 2)],
                out_hbm.at[pl.ds(f0, _FPW // 2)], out_sem)

        return carry

    per_feature(0, jnp.int32(0))
    for cp in rest:
        cp.wait()
    lax.fori_loop(1, _FPW, per_feature, jnp.int32(0))
    pltpu.async_copy(
        rows_v.at[pl.ds(_FPW // 2, _FPW // 2)],
        out_hbm.at[pl.ds(f0 + _FPW // 2, _FPW // 2)], out_sem)
    pltpu.make_async_copy(
        rows_v.at[pl.ds(0, _FPW // 2)],
        out_hbm.at[pl.ds(f0, _FPW // 2)], out_sem).wait()
    pltpu.make_async_copy(
        rows_v.at[pl.ds(_FPW // 2, _FPW // 2)],
        out_hbm.at[pl.ds(f0 + _FPW // 2, _FPW // 2)], out_sem).wait()


_impute_sc = functools.partial(
    pl.kernel,
    out_type=jax.ShapeDtypeStruct((D_M, ALPHA), jnp.bfloat16),
    mesh=plsc.VectorSubcoreMesh(
        core_axis_name="c", subcore_axis_name="s",
        num_cores=_NC, num_subcores=_NS),
    compiler_params=pltpu.CompilerParams(needs_layout_passes=False),
    scratch_types=[
        pltpu.VMEM((_FPW * L_OBS + _LANES,), jnp.int32),
        pltpu.VMEM((_FPW * L_OBS + _FPW,), jnp.float32),
        pltpu.VMEM((_FPW, ALPHA), jnp.bfloat16),
        pltpu.VMEM((ALPHA,), jnp.int32),
        pltpu.VMEM((ALPHA,), jnp.int32),
        pltpu.VMEM((_CHUNKS,), jnp.int32),
        pltpu.VMEM((_CHUNKS,), jnp.int32),
        pltpu.SemaphoreType.DMA,
        pltpu.SemaphoreType.DMA,
        pltpu.SemaphoreType.DMA,
    ],
)(_impute_body)


def _matmul_body(rs_ref, w_ref, b_ref, out_ref):
    out_ref[...] = lax.dot_general(
        rs_ref[...], w_ref[...].astype(jnp.bfloat16), (((0,), (1,)), ((), ())),
        preferred_element_type=jnp.float32) + b_ref[...]


_A_BLK = 1024


def _matmul_tc(rs, W, b2):
    return pl.pallas_call(
        _matmul_body,
        grid=(ALPHA // _A_BLK,),
        in_specs=[
            pl.BlockSpec((D_M, _A_BLK), lambda k: (0, k)),
            pl.BlockSpec((D_H, D_M), lambda k: (0, 0)),
            pl.BlockSpec((1, D_H), lambda k: (0, 0)),
        ],
        out_specs=pl.BlockSpec((_A_BLK, D_H), lambda k: (k, 0)),
        out_shape=jax.ShapeDtypeStruct((ALPHA, D_H), jnp.float32),
    )(rs, W, b2)


def kernel(x_ts, t_ts, global_means, W, b):
    rs = _impute_sc(t_ts.astype(jnp.int32), x_ts, global_means)
    return _matmul_tc(rs, W, b.reshape(1, D_H))
